# baseline (speedup 1.0000x reference)
import functools

import jax
import jax.numpy as jnp
import numpy as np
from jax import lax
from jax.experimental import pallas as pl
from jax.experimental.pallas import tpu as pltpu
from jax.experimental.pallas import tpu_sc as plsc

DIM = 16
B = 16384
L = 20
NCTX = 2 * L
NVOCAB = 1000000

NC = 2
NS = 16
NW = NC * NS

TCOLS = NVOCAB // 128
SWEEP_TC = 2
SWEEP_V = SWEEP_TC * 128
COLS_PW = TCOLS // NW
NSWEEP = COLS_PW // SWEEP_TC
XCOLS = TCOLS - COLS_PW * NW
TAIL_V = NVOCAB - TCOLS * 128
SCR_ROWS = NVOCAB // 8

BPW = B // NW
CB = 64
NCHUNK = BPW // CB
GROUPS = CB // 16
ROWS_PER_CHUNK = CB * NCTX
GATHER_ROWS = 128
IDX_ROWS = ROWS_PER_CHUNK // GATHER_ROWS
U_IDX_ROWS = BPW // GATHER_ROWS

_LANE = np.arange(16, dtype=np.int32)


def _relayout_body(ut_hbm, vt_hbm, utail_hbm, vtail_hbm, scru_hbm, scrv_hbm,
                   in_a, in_b, out_a, out_b, tail_v,
                   sem_ia, sem_ib, sem_oa, sem_ob):
    wid = lax.axis_index("s") * NC + lax.axis_index("c")
    lane = jnp.arange(16, dtype=jnp.int32)

    lane16 = lane * 16
    rcs = [(lane16 + d) >> 7 for d in range(DIM)]
    ccs = [(lane16 + d) & 127 for d in range(DIM)]

    def fetch(buf, sem, k):
        col = wid * COLS_PW + k * SWEEP_TC
        cu = pltpu.async_copy(
            ut_hbm.at[:, pl.ds(col * 128, SWEEP_V)],
            buf.at[:, pl.ds(0, SWEEP_V)], sem)
        cv = pltpu.async_copy(
            vt_hbm.at[:, pl.ds(col * 128, SWEEP_V)],
            buf.at[:, pl.ds(SWEEP_V, SWEEP_V)], sem)
        return cu, cv

    def flush(obuf, sem, k):
        col = wid * COLS_PW + k * SWEEP_TC
        cu = pltpu.async_copy(
            obuf.at[pl.ds(0, SWEEP_V // 8)],
            scru_hbm.at[pl.ds(col * 16, SWEEP_V // 8)], sem)
        cv = pltpu.async_copy(
            obuf.at[pl.ds(SWEEP_V // 8, SWEEP_V // 8)],
            scrv_hbm.at[pl.ds(col * 16, SWEEP_V // 8)], sem)
        return cu, cv

    def process(buf, obuf, k2):
        for g in range(SWEEP_V // 16):
            base = jnp.full((16,), g * 2, jnp.int32)
            for d in range(DIM):
                vec = buf[d, pl.ds(g * 16, 16)]
                plsc.store_scatter(obuf, [base + rcs[d], ccs[d]], vec)
            base2 = jnp.full((16,), SWEEP_V // 8 + g * 2, jnp.int32)
            for d in range(DIM):
                vec = buf[d, pl.ds(SWEEP_V + g * 16, 16)]
                plsc.store_scatter(obuf, [base2 + rcs[d], ccs[d]], vec)

    def wait_in(buf, sem):
        pltpu.make_async_copy(
            ut_hbm.at[:, pl.ds(0, SWEEP_V)],
            buf.at[:, pl.ds(0, SWEEP_V)], sem).wait()
        pltpu.make_async_copy(
            ut_hbm.at[:, pl.ds(0, SWEEP_V)],
            buf.at[:, pl.ds(SWEEP_V, SWEEP_V)], sem).wait()

    def wait_out(obuf, sem):
        pltpu.make_async_copy(
            obuf.at[pl.ds(0, SWEEP_V // 8)],
            scru_hbm.at[pl.ds(0, SWEEP_V // 8)], sem).wait()
        pltpu.make_async_copy(
            obuf.at[pl.ds(0, SWEEP_V // 8)],
            scru_hbm.at[pl.ds(0, SWEEP_V // 8)], sem).wait()

    fetch(in_a, sem_ia, 0)

    def sweep_body(k, carry):
        fetch(in_b, sem_ib, 2 * k + 1)
        wait_in(in_a, sem_ia)

        @pl.when(k > 0)
        def _():
            wait_out(out_a, sem_oa)

        process(in_a, out_a, 2 * k)
        flush(out_a, sem_oa, 2 * k)

        @pl.when(k < NSWEEP // 2 - 1)
        def _():
            fetch(in_a, sem_ia, 2 * k + 2)

        wait_in(in_b, sem_ib)

        @pl.when(k > 0)
        def _():
            wait_out(out_b, sem_ob)

        process(in_b, out_b, 2 * k + 1)
        flush(out_b, sem_ob, 2 * k + 1)
        return carry

    lax.fori_loop(0, NSWEEP // 2, sweep_body, 0)
    wait_out(out_a, sem_oa)
    wait_out(out_b, sem_ob)

    @pl.when(wid < XCOLS)
    def _():
        col = NW * COLS_PW + wid
        pltpu.sync_copy(ut_hbm.at[:, pl.ds(col * 128, 128)],
                        in_a.at[:, pl.ds(0, 128)])
        pltpu.sync_copy(vt_hbm.at[:, pl.ds(col * 128, 128)],
                        in_a.at[:, pl.ds(128, 128)])
        for t in range(2):
            for g in range(8):
                base = jnp.full((16,), t * 16 + g * 2, jnp.int32)
                for d in range(DIM):
                    vec = in_a[d, pl.ds(t * 128 + g * 16, 16)]
                    plsc.store_scatter(out_a, [base + rcs[d], ccs[d]], vec)
        pltpu.sync_copy(out_a.at[pl.ds(0, 16)],
                        scru_hbm.at[pl.ds(col * 16, 16)])
        pltpu.sync_copy(out_a.at[pl.ds(16, 16)],
                        scrv_hbm.at[pl.ds(col * 16, 16)])

    @pl.when(wid == NW - 1)
    def _():
        pltpu.sync_copy(utail_hbm, tail_v)
        pltpu.sync_copy(tail_v, scru_hbm.at[pl.ds(SCR_ROWS - 8, 8)])
        pltpu.sync_copy(vtail_hbm, tail_v)
        pltpu.sync_copy(tail_v, scrv_hbm.at[pl.ds(SCR_ROWS - 8, 8)])


def _score_body(ctx_idx_hbm, u_idx_hbm, u_emb_hbm, v_emb_hbm, out_hbm,
                ctx_idx_a, ctx_idx_b, u_idx_v, u_rows, ctx_rows_a,
                ctx_rows_b, acc_v, sem_u, sem_a, sem_b):
    wid = lax.axis_index("s") * NC + lax.axis_index("c")
    lane = jnp.arange(16, dtype=jnp.int32)
    idx_bufs = (ctx_idx_a, ctx_idx_b)
    row_bufs = (ctx_rows_a, ctx_rows_b)
    sems = (sem_a, sem_b)

    pltpu.sync_copy(u_idx_hbm.at[pl.ds(wid * BPW, BPW)], u_idx_v)
    ucps = [
        pltpu.async_copy(u_emb_hbm.at[u_idx_v.at[pl.ds(i * GATHER_ROWS,
                                                       GATHER_ROWS)]],
                         u_rows.at[pl.ds(i * GATHER_ROWS, GATHER_ROWS)],
                         sem_u)
        for i in range(U_IDX_ROWS)
    ]

    def fire_chunk(c):
        buf = c % 2
        pltpu.sync_copy(
            ctx_idx_hbm.at[pl.ds(wid * (BPW * NCTX) + c * ROWS_PER_CHUNK,
                                 ROWS_PER_CHUNK)],
            idx_bufs[buf])
        return [
            pltpu.async_copy(
                v_emb_hbm.at[idx_bufs[buf].at[pl.ds(i * GATHER_ROWS,
                                                    GATHER_ROWS)]],
                row_bufs[buf].at[pl.ds(i * GATHER_ROWS, GATHER_ROWS)],
                sems[buf])
            for i in range(IDX_ROWS)
        ]

    pending = fire_chunk(0)
    for cp in ucps:
        cp.wait()

    acc = jnp.zeros((16,), jnp.float32)
    for c in range(NCHUNK):
        nxt = fire_chunk(c + 1) if c + 1 < NCHUNK else []
        for cp in pending:
            cp.wait()
        pending = nxt
        ctx_rows = row_bufs[c % 2]

        for g in range(GROUPS):
            off = c * CB + g * 16
            u_lanes = [
                plsc.load_gather(
                    u_rows, [lane + off, jnp.full((16,), d, jnp.int32)])
                for d in range(DIM)
            ]
            base_rows = g * 16 * NCTX + lane * NCTX

            def jbody(j, acc, base_rows=base_rows, u_lanes=u_lanes,
                      ctx_rows=ctx_rows):
                rows = base_rows + j
                parts = [jnp.zeros((16,), jnp.float32) for _ in range(4)]
                for d in range(DIM):
                    cv = plsc.load_gather(
                        ctx_rows, [rows, jnp.full((16,), d, jnp.int32)])
                    parts[d & 3] = parts[d & 3] + cv * u_lanes[d]
                s = (parts[0] + parts[1]) + (parts[2] + parts[3])
                return acc + 1.0 / (1.0 + jnp.exp(s))

            acc = lax.fori_loop(0, NCTX, jbody, acc)

    acc_v[...] = acc
    pltpu.sync_copy(acc_v, out_hbm.at[pl.ds(wid * 16, 16)])


@jax.jit
def _sc_call(ctx_idx, u_idx, ut, vt, utail, vtail):
    mesh = plsc.VectorSubcoreMesh(core_axis_name="c", subcore_axis_name="s")
    relayout = functools.partial(
        pl.kernel,
        mesh=mesh,
        out_type=(jax.ShapeDtypeStruct((SCR_ROWS, 128), jnp.float32),
                  jax.ShapeDtypeStruct((SCR_ROWS, 128), jnp.float32)),
        scratch_types=[
            pltpu.VMEM((16, 2 * SWEEP_V), jnp.float32),
            pltpu.VMEM((16, 2 * SWEEP_V), jnp.float32),
            pltpu.VMEM((SWEEP_V // 4, 128), jnp.float32),
            pltpu.VMEM((SWEEP_V // 4, 128), jnp.float32),
            pltpu.VMEM((8, 128), jnp.float32),
            pltpu.SemaphoreType.DMA,
            pltpu.SemaphoreType.DMA,
            pltpu.SemaphoreType.DMA,
            pltpu.SemaphoreType.DMA,
        ],
        compiler_params=pltpu.CompilerParams(
            needs_layout_passes=False,
        ),
    )(_relayout_body)
    scru, scrv = relayout(ut, vt, utail, vtail)

    score = functools.partial(
        pl.kernel,
        mesh=mesh,
        out_type=jax.ShapeDtypeStruct((NW * 16,), jnp.float32),
        scratch_types=[
            pltpu.VMEM((ROWS_PER_CHUNK,), jnp.int32),
            pltpu.VMEM((ROWS_PER_CHUNK,), jnp.int32),
            pltpu.VMEM((BPW,), jnp.int32),
            pltpu.VMEM((BPW, DIM), jnp.float32),
            pltpu.VMEM((ROWS_PER_CHUNK, DIM), jnp.float32),
            pltpu.VMEM((ROWS_PER_CHUNK, DIM), jnp.float32),
            pltpu.VMEM((16,), jnp.float32),
            pltpu.SemaphoreType.DMA,
            pltpu.SemaphoreType.DMA,
            pltpu.SemaphoreType.DMA,
        ],
        compiler_params=pltpu.CompilerParams(
            needs_layout_passes=False,
            use_tc_tiling_on_sc=False,
        ),
    )(_score_body)
    return score(ctx_idx, u_idx,
                 scru.reshape(NVOCAB, DIM), scrv.reshape(NVOCAB, DIM))


def kernel(u, v, neg, u_emb_w, v_emb_w):
    u = u.astype(jnp.int32)
    ctx = jnp.concatenate([v.astype(jnp.int32), neg.astype(jnp.int32)],
                          axis=1)
    ctx_idx = ctx.reshape(B * NCTX)
    utail = u_emb_w[TCOLS * 128:].reshape(8, 128)
    vtail = v_emb_w[TCOLS * 128:].reshape(8, 128)
    partial = _sc_call(ctx_idx, u, u_emb_w.T, v_emb_w.T, utail, vtail)
    return jnp.sum(partial) / (B * NCTX)


# speedup vs baseline: 1.0051x; 1.0051x over previous
import functools

import jax
import jax.numpy as jnp
import numpy as np
from jax import lax
from jax.experimental import pallas as pl
from jax.experimental.pallas import tpu as pltpu
from jax.experimental.pallas import tpu_sc as plsc

DIM = 16
B = 16384
L = 20
NCTX = 2 * L
NVOCAB = 1000000

NC = 2
NS = 16
NW = NC * NS

TCOLS = NVOCAB // 128
SWEEP_TC = 2
SWEEP_V = SWEEP_TC * 128
COLS_PW = TCOLS // NW
NSWEEP = COLS_PW // SWEEP_TC
XCOLS = TCOLS - COLS_PW * NW
TAIL_V = NVOCAB - TCOLS * 128
SCR_ROWS = NVOCAB // 8

BPW = B // NW
CB = 64
NCHUNK = BPW // CB
GROUPS = CB // 16
ROWS_PER_CHUNK = CB * NCTX
GATHER_ROWS = 640
IDX_ROWS = ROWS_PER_CHUNK // GATHER_ROWS
U_IDX_ROWS = 1

_LANE = np.arange(16, dtype=np.int32)


def _relayout_body(ut_hbm, vt_hbm, utail_hbm, vtail_hbm, scru_hbm, scrv_hbm,
                   in_a, in_b, out_a, out_b, tail_v,
                   sem_ia, sem_ib, sem_oa, sem_ob):
    wid = lax.axis_index("s") * NC + lax.axis_index("c")
    lane = jnp.arange(16, dtype=jnp.int32)

    lane16 = lane * 16
    rc = lane16 >> 7
    rcs = [rc for _ in range(DIM)]
    ccs = [(lane16 & 127) + d for d in range(DIM)]

    def fetch(buf, sem, k):
        col = wid * COLS_PW + k * SWEEP_TC
        cu = pltpu.async_copy(
            ut_hbm.at[:, pl.ds(col * 128, SWEEP_V)],
            buf.at[:, pl.ds(0, SWEEP_V)], sem)
        cv = pltpu.async_copy(
            vt_hbm.at[:, pl.ds(col * 128, SWEEP_V)],
            buf.at[:, pl.ds(SWEEP_V, SWEEP_V)], sem)
        return cu, cv

    def flush(obuf, sem, k):
        col = wid * COLS_PW + k * SWEEP_TC
        cu = pltpu.async_copy(
            obuf.at[pl.ds(0, SWEEP_V // 8)],
            scru_hbm.at[pl.ds(col * 16, SWEEP_V // 8)], sem)
        cv = pltpu.async_copy(
            obuf.at[pl.ds(SWEEP_V // 8, SWEEP_V // 8)],
            scrv_hbm.at[pl.ds(col * 16, SWEEP_V // 8)], sem)
        return cu, cv

    def process(buf, obuf, k2):
        for g in range(SWEEP_V // 16):
            rows = rc + (g * 2)
            for d in range(DIM):
                vec = buf[d, pl.ds(g * 16, 16)]
                plsc.store_scatter(obuf, [rows, ccs[d]], vec)
            rows2 = rc + (SWEEP_V // 8 + g * 2)
            for d in range(DIM):
                vec = buf[d, pl.ds(SWEEP_V + g * 16, 16)]
                plsc.store_scatter(obuf, [rows2, ccs[d]], vec)

    def wait_in(buf, sem):
        pltpu.make_async_copy(
            ut_hbm.at[:, pl.ds(0, SWEEP_V)],
            buf.at[:, pl.ds(0, SWEEP_V)], sem).wait()
        pltpu.make_async_copy(
            ut_hbm.at[:, pl.ds(0, SWEEP_V)],
            buf.at[:, pl.ds(SWEEP_V, SWEEP_V)], sem).wait()

    def wait_out(obuf, sem):
        pltpu.make_async_copy(
            obuf.at[pl.ds(0, SWEEP_V // 8)],
            scru_hbm.at[pl.ds(0, SWEEP_V // 8)], sem).wait()
        pltpu.make_async_copy(
            obuf.at[pl.ds(0, SWEEP_V // 8)],
            scru_hbm.at[pl.ds(0, SWEEP_V // 8)], sem).wait()

    fetch(in_a, sem_ia, 0)

    def sweep_body(k, carry):
        fetch(in_b, sem_ib, 2 * k + 1)
        wait_in(in_a, sem_ia)

        @pl.when(k > 0)
        def _():
            wait_out(out_a, sem_oa)

        process(in_a, out_a, 2 * k)
        flush(out_a, sem_oa, 2 * k)

        @pl.when(k < NSWEEP // 2 - 1)
        def _():
            fetch(in_a, sem_ia, 2 * k + 2)

        wait_in(in_b, sem_ib)

        @pl.when(k > 0)
        def _():
            wait_out(out_b, sem_ob)

        process(in_b, out_b, 2 * k + 1)
        flush(out_b, sem_ob, 2 * k + 1)
        return carry

    lax.fori_loop(0, NSWEEP // 2, sweep_body, 0)
    wait_out(out_a, sem_oa)
    wait_out(out_b, sem_ob)

    @pl.when(wid < XCOLS)
    def _():
        col = NW * COLS_PW + wid
        pltpu.sync_copy(ut_hbm.at[:, pl.ds(col * 128, 128)],
                        in_a.at[:, pl.ds(0, 128)])
        pltpu.sync_copy(vt_hbm.at[:, pl.ds(col * 128, 128)],
                        in_a.at[:, pl.ds(128, 128)])
        for t in range(2):
            for g in range(8):
                base = jnp.full((16,), t * 16 + g * 2, jnp.int32)
                for d in range(DIM):
                    vec = in_a[d, pl.ds(t * 128 + g * 16, 16)]
                    plsc.store_scatter(out_a, [base + rcs[d], ccs[d]], vec)
        pltpu.sync_copy(out_a.at[pl.ds(0, 16)],
                        scru_hbm.at[pl.ds(col * 16, 16)])
        pltpu.sync_copy(out_a.at[pl.ds(16, 16)],
                        scrv_hbm.at[pl.ds(col * 16, 16)])

    @pl.when(wid == NW - 1)
    def _():
        pltpu.sync_copy(utail_hbm, tail_v)
        pltpu.sync_copy(tail_v, scru_hbm.at[pl.ds(SCR_ROWS - 8, 8)])
        pltpu.sync_copy(vtail_hbm, tail_v)
        pltpu.sync_copy(tail_v, scrv_hbm.at[pl.ds(SCR_ROWS - 8, 8)])


def _score_body(ctx_idx_hbm, u_idx_hbm, u_emb_hbm, v_emb_hbm, out_hbm,
                ctx_idx_a, ctx_idx_b, u_idx_v, u_rows, ctx_rows_a,
                ctx_rows_b, acc_v, sem_u, sem_a, sem_b):
    wid = lax.axis_index("s") * NC + lax.axis_index("c")
    lane = jnp.arange(16, dtype=jnp.int32)
    idx_bufs = (ctx_idx_a, ctx_idx_b)
    row_bufs = (ctx_rows_a, ctx_rows_b)
    sems = (sem_a, sem_b)

    pltpu.sync_copy(u_idx_hbm.at[pl.ds(wid * BPW, BPW)], u_idx_v)
    ucps = [pltpu.async_copy(u_emb_hbm.at[u_idx_v], u_rows, sem_u)]

    def fire_chunk(c):
        buf = c % 2
        pltpu.sync_copy(
            ctx_idx_hbm.at[pl.ds(wid * (BPW * NCTX) + c * ROWS_PER_CHUNK,
                                 ROWS_PER_CHUNK)],
            idx_bufs[buf])
        return [
            pltpu.async_copy(
                v_emb_hbm.at[idx_bufs[buf].at[pl.ds(i * GATHER_ROWS,
                                                    GATHER_ROWS)]],
                row_bufs[buf].at[pl.ds(i * GATHER_ROWS, GATHER_ROWS)],
                sems[buf])
            for i in range(IDX_ROWS)
        ]

    pending = fire_chunk(0)
    for cp in ucps:
        cp.wait()

    acc = jnp.zeros((16,), jnp.float32)
    for c in range(NCHUNK):
        nxt = fire_chunk(c + 1) if c + 1 < NCHUNK else []
        for cp in pending:
            cp.wait()
        pending = nxt
        ctx_rows = row_bufs[c % 2]

        for g in range(GROUPS):
            off = c * CB + g * 16
            u_lanes = [
                plsc.load_gather(
                    u_rows, [lane + off, jnp.full((16,), d, jnp.int32)])
                for d in range(DIM)
            ]
            base_rows = g * 16 * NCTX + lane * NCTX

            def jbody(j, acc, base_rows=base_rows, u_lanes=u_lanes,
                      ctx_rows=ctx_rows):
                rows = base_rows + j
                parts = [jnp.zeros((16,), jnp.float32) for _ in range(4)]
                for d in range(DIM):
                    cv = plsc.load_gather(
                        ctx_rows, [rows, jnp.full((16,), d, jnp.int32)])
                    parts[d & 3] = parts[d & 3] + cv * u_lanes[d]
                s = (parts[0] + parts[1]) + (parts[2] + parts[3])
                return acc + 1.0 / (1.0 + jnp.exp(s))

            acc = lax.fori_loop(0, NCTX, jbody, acc)

    acc_v[...] = acc
    pltpu.sync_copy(acc_v, out_hbm.at[pl.ds(wid * 16, 16)])


@jax.jit
def _sc_call(ctx_idx, u_idx, ut, vt, utail, vtail):
    mesh = plsc.VectorSubcoreMesh(core_axis_name="c", subcore_axis_name="s")
    relayout = functools.partial(
        pl.kernel,
        mesh=mesh,
        out_type=(jax.ShapeDtypeStruct((SCR_ROWS, 128), jnp.float32),
                  jax.ShapeDtypeStruct((SCR_ROWS, 128), jnp.float32)),
        scratch_types=[
            pltpu.VMEM((16, 2 * SWEEP_V), jnp.float32),
            pltpu.VMEM((16, 2 * SWEEP_V), jnp.float32),
            pltpu.VMEM((SWEEP_V // 4, 128), jnp.float32),
            pltpu.VMEM((SWEEP_V // 4, 128), jnp.float32),
            pltpu.VMEM((8, 128), jnp.float32),
            pltpu.SemaphoreType.DMA,
            pltpu.SemaphoreType.DMA,
            pltpu.SemaphoreType.DMA,
            pltpu.SemaphoreType.DMA,
        ],
        compiler_params=pltpu.CompilerParams(
            needs_layout_passes=False,
        ),
    )(_relayout_body)
    scru, scrv = relayout(ut, vt, utail, vtail)

    score = functools.partial(
        pl.kernel,
        mesh=mesh,
        out_type=jax.ShapeDtypeStruct((NW * 16,), jnp.float32),
        scratch_types=[
            pltpu.VMEM((ROWS_PER_CHUNK,), jnp.int32),
            pltpu.VMEM((ROWS_PER_CHUNK,), jnp.int32),
            pltpu.VMEM((BPW,), jnp.int32),
            pltpu.VMEM((BPW, DIM), jnp.float32),
            pltpu.VMEM((ROWS_PER_CHUNK, DIM), jnp.float32),
            pltpu.VMEM((ROWS_PER_CHUNK, DIM), jnp.float32),
            pltpu.VMEM((16,), jnp.float32),
            pltpu.SemaphoreType.DMA,
            pltpu.SemaphoreType.DMA,
            pltpu.SemaphoreType.DMA,
        ],
        compiler_params=pltpu.CompilerParams(
            needs_layout_passes=False,
            use_tc_tiling_on_sc=False,
        ),
    )(_score_body)
    return score(ctx_idx, u_idx,
                 scru.reshape(NVOCAB, DIM), scrv.reshape(NVOCAB, DIM))


def kernel(u, v, neg, u_emb_w, v_emb_w):
    u = u.astype(jnp.int32)
    ctx = jnp.concatenate([v.astype(jnp.int32), neg.astype(jnp.int32)],
                          axis=1)
    ctx_idx = ctx.reshape(B * NCTX)
    utail = u_emb_w[TCOLS * 128:].reshape(8, 128)
    vtail = v_emb_w[TCOLS * 128:].reshape(8, 128)
    partial = _sc_call(ctx_idx, u, u_emb_w.T, v_emb_w.T, utail, vtail)
    return jnp.sum(partial) / (B * NCTX)


# speedup vs baseline: 1.0086x; 1.0034x over previous
import functools

import jax
import jax.numpy as jnp
from jax import lax
from jax.experimental import pallas as pl
from jax.experimental.pallas import tpu as pltpu
from jax.experimental.pallas import tpu_sc as plsc

DIM = 16
B = 16384
L = 20
NCTX = 2 * L
NVOCAB = 1000000

NC = 2
NS = 16
NW = NC * NS

TCOLS = NVOCAB // 128
SWEEP_TC = 2
SWEEP_V = SWEEP_TC * 128
COLS_PW = TCOLS // NW
NSWEEP = COLS_PW // SWEEP_TC
XCOLS = TCOLS - COLS_PW * NW
TAIL_V = NVOCAB - TCOLS * 128
SCR_ROWS = NVOCAB // 8

BPW = B // NW
CB = 64
NCHUNK = BPW // CB
GROUPS = CB // 16
ROWS_PER_CHUNK = CB * NCTX
GATHER_ROWS = 640
IDX_ROWS = ROWS_PER_CHUNK // GATHER_ROWS
U_IDX_ROWS = 1


def _relayout_body(ut_hbm, vt_hbm, utail_hbm, vtail_hbm, scru_hbm, scrv_hbm,
                   in_a, in_b, out_a, out_b, tail_v,
                   sem_ia, sem_ib, sem_oa, sem_ob):
    wid = lax.axis_index("s") * NC + lax.axis_index("c")
    lane = jnp.arange(16, dtype=jnp.int32)

    lane16 = lane * 16
    rc = lane16 >> 7
    rcs = [rc for _ in range(DIM)]
    ccs = [(lane16 & 127) + d for d in range(DIM)]

    def fetch(buf, sem, k):
        col = wid * COLS_PW + k * SWEEP_TC
        cu = pltpu.async_copy(
            ut_hbm.at[:, pl.ds(col * 128, SWEEP_V)],
            buf.at[:, pl.ds(0, SWEEP_V)], sem)
        cv = pltpu.async_copy(
            vt_hbm.at[:, pl.ds(col * 128, SWEEP_V)],
            buf.at[:, pl.ds(SWEEP_V, SWEEP_V)], sem)
        return cu, cv

    def flush(obuf, sem, k):
        col = wid * COLS_PW + k * SWEEP_TC
        cu = pltpu.async_copy(
            obuf.at[pl.ds(0, SWEEP_V // 8)],
            scru_hbm.at[pl.ds(col * 16, SWEEP_V // 8)], sem)
        cv = pltpu.async_copy(
            obuf.at[pl.ds(SWEEP_V // 8, SWEEP_V // 8)],
            scrv_hbm.at[pl.ds(col * 16, SWEEP_V // 8)], sem)
        return cu, cv

    def process(buf, obuf):
        for g in range(SWEEP_V // 16):
            rows = rc + (g * 2)
            for d in range(DIM):
                vec = buf[d, pl.ds(g * 16, 16)]
                plsc.store_scatter(obuf, [rows, ccs[d]], vec)
            rows2 = rc + (SWEEP_V // 8 + g * 2)
            for d in range(DIM):
                vec = buf[d, pl.ds(SWEEP_V + g * 16, 16)]
                plsc.store_scatter(obuf, [rows2, ccs[d]], vec)

    def wait_in(buf, sem):
        pltpu.make_async_copy(
            ut_hbm.at[:, pl.ds(0, SWEEP_V)],
            buf.at[:, pl.ds(0, SWEEP_V)], sem).wait()
        pltpu.make_async_copy(
            ut_hbm.at[:, pl.ds(0, SWEEP_V)],
            buf.at[:, pl.ds(SWEEP_V, SWEEP_V)], sem).wait()

    def wait_out(obuf, sem):
        pltpu.make_async_copy(
            obuf.at[pl.ds(0, SWEEP_V // 8)],
            scru_hbm.at[pl.ds(0, SWEEP_V // 8)], sem).wait()
        pltpu.make_async_copy(
            obuf.at[pl.ds(0, SWEEP_V // 8)],
            scru_hbm.at[pl.ds(0, SWEEP_V // 8)], sem).wait()

    fetch(in_a, sem_ia, 0)

    def sweep_body(k, carry):
        fetch(in_b, sem_ib, 2 * k + 1)
        wait_in(in_a, sem_ia)

        @pl.when(k > 0)
        def _():
            wait_out(out_a, sem_oa)

        process(in_a, out_a)
        flush(out_a, sem_oa, 2 * k)

        @pl.when(k < NSWEEP // 2 - 1)
        def _():
            fetch(in_a, sem_ia, 2 * k + 2)

        wait_in(in_b, sem_ib)

        @pl.when(k > 0)
        def _():
            wait_out(out_b, sem_ob)

        process(in_b, out_b)
        flush(out_b, sem_ob, 2 * k + 1)
        return carry

    lax.fori_loop(0, NSWEEP // 2, sweep_body, 0)
    wait_out(out_a, sem_oa)
    wait_out(out_b, sem_ob)

    @pl.when(wid < XCOLS)
    def _():
        col = NW * COLS_PW + wid
        pltpu.sync_copy(ut_hbm.at[:, pl.ds(col * 128, 128)],
                        in_a.at[:, pl.ds(0, 128)])
        pltpu.sync_copy(vt_hbm.at[:, pl.ds(col * 128, 128)],
                        in_a.at[:, pl.ds(128, 128)])
        for t in range(2):
            for g in range(8):
                base = jnp.full((16,), t * 16 + g * 2, jnp.int32)
                for d in range(DIM):
                    vec = in_a[d, pl.ds(t * 128 + g * 16, 16)]
                    plsc.store_scatter(out_a, [base + rcs[d], ccs[d]], vec)
        pltpu.sync_copy(out_a.at[pl.ds(0, 16)],
                        scru_hbm.at[pl.ds(col * 16, 16)])
        pltpu.sync_copy(out_a.at[pl.ds(16, 16)],
                        scrv_hbm.at[pl.ds(col * 16, 16)])

    @pl.when(wid == NW - 1)
    def _():
        pltpu.sync_copy(utail_hbm, tail_v)
        pltpu.sync_copy(tail_v, scru_hbm.at[pl.ds(SCR_ROWS - 8, 8)])
        pltpu.sync_copy(vtail_hbm, tail_v)
        pltpu.sync_copy(tail_v, scrv_hbm.at[pl.ds(SCR_ROWS - 8, 8)])


def _score_body(ctx_idx_hbm, u_idx_hbm, u_emb_hbm, v_emb_hbm, out_hbm,
                ctx_idx_a, ctx_idx_b, u_idx_v, u_rows, ctx_rows_a,
                ctx_rows_b, acc_v, sem_u, sem_a, sem_b):
    wid = lax.axis_index("s") * NC + lax.axis_index("c")
    lane = jnp.arange(16, dtype=jnp.int32)
    idx_bufs = (ctx_idx_a, ctx_idx_b)
    row_bufs = (ctx_rows_a, ctx_rows_b)
    sems = (sem_a, sem_b)

    pltpu.sync_copy(u_idx_hbm.at[pl.ds(wid * BPW, BPW)], u_idx_v)
    ucps = [pltpu.async_copy(u_emb_hbm.at[u_idx_v], u_rows, sem_u)]

    def fire_chunk(c):
        buf = c % 2
        pltpu.sync_copy(
            ctx_idx_hbm.at[pl.ds(wid * (BPW * NCTX) + c * ROWS_PER_CHUNK,
                                 ROWS_PER_CHUNK)],
            idx_bufs[buf])
        return [
            pltpu.async_copy(
                v_emb_hbm.at[idx_bufs[buf].at[pl.ds(i * GATHER_ROWS,
                                                    GATHER_ROWS)]],
                row_bufs[buf].at[pl.ds(i * GATHER_ROWS, GATHER_ROWS)],
                sems[buf])
            for i in range(IDX_ROWS)
        ]

    pending = fire_chunk(0)
    for cp in ucps:
        cp.wait()

    acc = jnp.zeros((16,), jnp.float32)
    for c in range(NCHUNK):
        nxt = fire_chunk(c + 1) if c + 1 < NCHUNK else []
        for cp in pending:
            cp.wait()
        pending = nxt
        ctx_rows = row_bufs[c % 2]

        for g in range(GROUPS):
            off = c * CB + g * 16
            u_lanes = [
                plsc.load_gather(
                    u_rows, [lane + off, jnp.full((16,), d, jnp.int32)])
                for d in range(DIM)
            ]
            base_rows = g * 16 * NCTX + lane * NCTX

            def jbody(j, acc, base_rows=base_rows, u_lanes=u_lanes,
                      ctx_rows=ctx_rows):
                rows = base_rows + j
                parts = [jnp.zeros((16,), jnp.float32) for _ in range(4)]
                for d in range(DIM):
                    cv = plsc.load_gather(
                        ctx_rows, [rows, jnp.full((16,), d, jnp.int32)])
                    parts[d & 3] = parts[d & 3] + cv * u_lanes[d]
                s = (parts[0] + parts[1]) + (parts[2] + parts[3])
                return acc + 1.0 / (1.0 + jnp.exp(s))

            acc = lax.fori_loop(0, NCTX, jbody, acc)

    acc_v[...] = acc
    pltpu.sync_copy(acc_v, out_hbm.at[pl.ds(wid * 16, 16)])


@jax.jit
def _sc_call(ctx_idx, u_idx, ut, vt, utail, vtail):
    mesh = plsc.VectorSubcoreMesh(core_axis_name="c", subcore_axis_name="s")
    relayout = functools.partial(
        pl.kernel,
        mesh=mesh,
        out_type=(jax.ShapeDtypeStruct((SCR_ROWS, 128), jnp.float32),
                  jax.ShapeDtypeStruct((SCR_ROWS, 128), jnp.float32)),
        scratch_types=[
            pltpu.VMEM((16, 2 * SWEEP_V), jnp.float32),
            pltpu.VMEM((16, 2 * SWEEP_V), jnp.float32),
            pltpu.VMEM((SWEEP_V // 4, 128), jnp.float32),
            pltpu.VMEM((SWEEP_V // 4, 128), jnp.float32),
            pltpu.VMEM((8, 128), jnp.float32),
            pltpu.SemaphoreType.DMA,
            pltpu.SemaphoreType.DMA,
            pltpu.SemaphoreType.DMA,
            pltpu.SemaphoreType.DMA,
        ],
        compiler_params=pltpu.CompilerParams(
            needs_layout_passes=False,
        ),
    )(_relayout_body)
    scru, scrv = relayout(ut, vt, utail, vtail)

    score = functools.partial(
        pl.kernel,
        mesh=mesh,
        out_type=jax.ShapeDtypeStruct((NW * 16,), jnp.float32),
        scratch_types=[
            pltpu.VMEM((ROWS_PER_CHUNK,), jnp.int32),
            pltpu.VMEM((ROWS_PER_CHUNK,), jnp.int32),
            pltpu.VMEM((BPW,), jnp.int32),
            pltpu.VMEM((BPW, DIM), jnp.float32),
            pltpu.VMEM((ROWS_PER_CHUNK, DIM), jnp.float32),
            pltpu.VMEM((ROWS_PER_CHUNK, DIM), jnp.float32),
            pltpu.VMEM((16,), jnp.float32),
            pltpu.SemaphoreType.DMA,
            pltpu.SemaphoreType.DMA,
            pltpu.SemaphoreType.DMA,
        ],
        compiler_params=pltpu.CompilerParams(
            needs_layout_passes=False,
            use_tc_tiling_on_sc=False,
        ),
    )(_score_body)
    return score(ctx_idx, u_idx,
                 scru.reshape(NVOCAB, DIM), scrv.reshape(NVOCAB, DIM))


def kernel(u, v, neg, u_emb_w, v_emb_w):
    u = u.astype(jnp.int32)
    ctx = jnp.concatenate([v.astype(jnp.int32), neg.astype(jnp.int32)],
                          axis=1)
    ctx_idx = ctx.reshape(B * NCTX)
    utail = u_emb_w[TCOLS * 128:].reshape(8, 128)
    vtail = v_emb_w[TCOLS * 128:].reshape(8, 128)
    partial = _sc_call(ctx_idx, u, u_emb_w.T, v_emb_w.T, utail, vtail)
    return jnp.sum(partial) / (B * NCTX)
